# baseline (speedup 1.0000x reference)
import functools

import jax
import jax.numpy as jnp
from jax import lax
from jax.experimental import pallas as pl
from jax.experimental.pallas import tpu as pltpu
from jax.experimental.pallas import tpu_sc as plsc

N_USERS = 100000
N_ITEMS = 1000000
HIDDEN = 64
BATCH = 16384

_NC = 2
_NS = 16
_NW = _NC * _NS
_BPW = BATCH // _NW
_L = 16
_HB = _BPW // 2
_NGH = _HB // _L


def _bpr_body(users_hbm, items_hbm, ua_hbm, ia_hbm, ub_hbm, ib_hbm, out_hbm,
              ur_v, ir_v, upr_v, ipr_v, ug_v, ig_v, ub_v, ib_v, out_v,
              sem0, sem1, sem2, sem3):
    wid = lax.axis_index("s") * _NC + lax.axis_index("c")
    base = wid * _BPW

    pltpu.sync_copy(users_hbm.at[pl.ds(base, _BPW)], ur_v)
    pltpu.sync_copy(items_hbm.at[pl.ds(base, _BPW)], ir_v)

    cb0 = pltpu.async_copy(ub_hbm.at[ur_v], ub_v, sem2)
    cb1 = pltpu.async_copy(ib_hbm.at[ir_v], ib_v, sem3)

    def pair_step(g, carry):
        upr_v[pl.ds(g * _L, _L)] = ur_v[pl.ds(g * _L, _L)] >> 1
        ipr_v[pl.ds(g * _L, _L)] = ir_v[pl.ds(g * _L, _L)] >> 1
        return carry

    lax.fori_loop(0, _BPW // _L, pair_step, 0)
    cb0.wait()
    cb1.wait()

    def half(half_i, carry):
        h0 = half_i * _HB
        c0 = pltpu.async_copy(ua_hbm.at[upr_v.at[pl.ds(h0, _HB)]], ug_v, sem0)
        c1 = pltpu.async_copy(ia_hbm.at[ipr_v.at[pl.ds(h0, _HB)]], ig_v, sem1)
        c0.wait()
        c1.wait()

        def group(g, carry2):
            r0 = g * _L
            rows = lax.iota(jnp.int32, _L)
            acc = ub_v[pl.ds(h0 + r0, _L)] + ib_v[pl.ds(h0 + r0, _L)]
            ucol0 = (ur_v[pl.ds(h0 + r0, _L)] & 1) << 6
            icol0 = (ir_v[pl.ds(h0 + r0, _L)] & 1) << 6
            rows = rows + r0

            def hstep(h, a):
                uv = plsc.load_gather(ug_v, [rows, ucol0 + h])
                iv = plsc.load_gather(ig_v, [rows, icol0 + h])
                return a + uv * iv

            acc = lax.fori_loop(0, HIDDEN, hstep, acc)
            out_v[pl.ds(h0 + r0, _L)] = acc
            return carry2

        lax.fori_loop(0, _NGH, group, 0)
        return carry

    lax.fori_loop(0, 2, half, 0)
    pltpu.sync_copy(out_v, out_hbm.at[pl.ds(base, _BPW)])


@jax.jit
def _bpr(users, items, ua2, ia2, user_beta, item_beta):
    mesh = plsc.VectorSubcoreMesh(core_axis_name="c", subcore_axis_name="s")
    run = functools.partial(
        pl.kernel,
        mesh=mesh,
        compiler_params=pltpu.CompilerParams(needs_layout_passes=False),
        out_type=jax.ShapeDtypeStruct((BATCH,), jnp.float32),
        scratch_types=[
            pltpu.VMEM((_BPW,), jnp.int32),
            pltpu.VMEM((_BPW,), jnp.int32),
            pltpu.VMEM((_BPW,), jnp.int32),
            pltpu.VMEM((_BPW,), jnp.int32),
            pltpu.VMEM((_HB, 2 * HIDDEN), jnp.float32),
            pltpu.VMEM((_HB, 2 * HIDDEN), jnp.float32),
            pltpu.VMEM((_BPW,), jnp.float32),
            pltpu.VMEM((_BPW,), jnp.float32),
            pltpu.VMEM((_BPW,), jnp.float32),
            pltpu.SemaphoreType.DMA,
            pltpu.SemaphoreType.DMA,
            pltpu.SemaphoreType.DMA,
            pltpu.SemaphoreType.DMA,
        ],
    )(_bpr_body)
    return run(users, items, ua2, ia2, user_beta, item_beta)


def kernel(users, items, user_alpha, item_alpha, user_beta, item_beta):
    users = users.astype(jnp.int32)
    items = items.astype(jnp.int32)
    ua2 = user_alpha.reshape(N_USERS // 2, 2 * HIDDEN)
    ia2 = item_alpha.reshape(N_ITEMS // 2, 2 * HIDDEN)
    ub = user_beta.reshape(-1)
    ib = item_beta.reshape(-1)
    return _bpr(users, items, ua2, ia2, ub, ib)


# speedup vs baseline: 2.3598x; 2.3598x over previous
import functools

import jax
import jax.numpy as jnp
from jax import lax
from jax.experimental import pallas as pl
from jax.experimental.pallas import tpu as pltpu
from jax.experimental.pallas import tpu_sc as plsc

N_USERS = 100000
N_ITEMS = 1000000
HIDDEN = 64
BATCH = 16384

_NC = 2
_NS = 16
_NW = _NC * _NS
_BPW = BATCH // _NW
_L = 16
_CH = 32
_NCH = _BPW // _CH
_CW = _CH * HIDDEN


def _bpr_body(users_hbm, items_hbm, ua_hbm, ia_hbm, ub_hbm, ib_hbm, out_hbm,
              ur_v, ir_v, ug_v, ig_v, ub_v, ib_v, out_v,
              semu, semi, semb0, semb1):
    wid = lax.axis_index("s") * _NC + lax.axis_index("c")
    base = wid * _BPW

    pltpu.sync_copy(users_hbm.at[pl.ds(base, _BPW)], ur_v)
    pltpu.sync_copy(items_hbm.at[pl.ds(base, _BPW)], ir_v)

    cb0 = pltpu.async_copy(ub_hbm.at[ur_v], ub_v, semb0)
    cb1 = pltpu.async_copy(ib_hbm.at[ir_v], ib_v, semb1)

    def issue(c, slot):
        for g in range(_CH // _L):
            uvec = ur_v[pl.ds(c * _CH + g * _L, _L)]
            ivec = ir_v[pl.ds(c * _CH + g * _L, _L)]
            for j in range(_L):
                d = g * _L + j
                pltpu.async_copy(ua_hbm.at[uvec[j] >> 3, uvec[j] & 7],
                                 ug_v.at[slot, d], semu)
                pltpu.async_copy(ia_hbm.at[ivec[j] >> 3, ivec[j] & 7],
                                 ig_v.at[slot, d], semi)

    def drain(slot):
        for k in range(_CH // 8):
            pltpu.make_async_copy(
                ua_hbm.at[0], ug_v.at[slot, pl.ds(8 * k, 8), :], semu).wait()
            pltpu.make_async_copy(
                ia_hbm.at[0], ig_v.at[slot, pl.ds(8 * k, 8), :], semi).wait()

    issue(0, 0)
    cb0.wait()
    cb1.wait()

    def chunk_step(c, carry):
        slot = lax.rem(c, 2)

        @pl.when(c + 1 < _NCH)
        def _():
            issue(c + 1, 1 - slot)

        drain(slot)

        def group(g, carry2):
            r0 = c * _CH + g * _L
            acc = ub_v[pl.ds(r0, _L)] + ib_v[pl.ds(r0, _L)]
            slots = jnp.full((_L,), slot, jnp.int32)
            rows = lax.iota(jnp.int32, _L) + g * _L

            def hstep(h, a):
                cols = jnp.full((_L,), h, jnp.int32)
                uv = plsc.load_gather(ug_v, [slots, rows, cols])
                iv = plsc.load_gather(ig_v, [slots, rows, cols])
                return a + uv * iv

            acc = lax.fori_loop(0, HIDDEN, hstep, acc)
            out_v[pl.ds(r0, _L)] = acc
            return carry2

        lax.fori_loop(0, _CH // _L, group, 0)
        return carry

    lax.fori_loop(0, _NCH, chunk_step, 0)
    pltpu.sync_copy(out_v, out_hbm.at[pl.ds(base, _BPW)])


@jax.jit
def _bpr(users, items, ua3, ia3, user_beta, item_beta):
    mesh = plsc.VectorSubcoreMesh(core_axis_name="c", subcore_axis_name="s")
    run = functools.partial(
        pl.kernel,
        mesh=mesh,
        compiler_params=pltpu.CompilerParams(
            needs_layout_passes=False, use_tc_tiling_on_sc=True),
        out_type=jax.ShapeDtypeStruct((BATCH,), jnp.float32),
        scratch_types=[
            pltpu.VMEM((_BPW,), jnp.int32),
            pltpu.VMEM((_BPW,), jnp.int32),
            pltpu.VMEM((2, _CH, HIDDEN), jnp.float32),
            pltpu.VMEM((2, _CH, HIDDEN), jnp.float32),
            pltpu.VMEM((_BPW,), jnp.float32),
            pltpu.VMEM((_BPW,), jnp.float32),
            pltpu.VMEM((_BPW,), jnp.float32),
            pltpu.SemaphoreType.DMA,
            pltpu.SemaphoreType.DMA,
            pltpu.SemaphoreType.DMA,
            pltpu.SemaphoreType.DMA,
        ],
    )(_bpr_body)
    return run(users, items, ua3, ia3, user_beta, item_beta)


def kernel(users, items, user_alpha, item_alpha, user_beta, item_beta):
    users = users.astype(jnp.int32)
    items = items.astype(jnp.int32)
    ua3 = user_alpha.reshape(N_USERS // 8, 8, HIDDEN)
    ia3 = item_alpha.reshape(N_ITEMS // 8, 8, HIDDEN)
    ub = user_beta.reshape(-1)
    ib = item_beta.reshape(-1)
    return _bpr(users, items, ua3, ia3, ub, ib)
